# baseline (speedup 1.0000x reference)
import dataclasses
import functools

import jax
import jax.numpy as jnp
from jax import lax
from jax.experimental import pallas as pl
from jax.experimental.pallas import tpu as pltpu
from jax.experimental.pallas import tpu_sc as plsc

N = 10000
E = 320000
D_IN = 128
H1 = 8
C1 = 8
D1 = H1 * C1
D2 = 128
G = 16

NC, NS, L = 2, 16, 16
NW = NC * NS
ROWB = 1000
GRID = N // ROWB
RCH = 40
NCH = N // RCH

F32 = jnp.float32
I32 = jnp.int32

_GDN = lax.GatherDimensionNumbers(
    offset_dims=(), collapsed_slice_dims=(0,), start_index_map=(0,))


def _dyn_gather(v, idx):
    return lax.gather(v, idx[:, None], _GDN, slice_sizes=(1,),
                      mode=lax.GatherScatterMode.PROMISE_IN_BOUNDS)


def _dense_body(x_ref, w_ref, as_ref, ad_ref, h_ref, p_ref, q_ref,
                pm_ref, qm_ref):
    h = jnp.dot(x_ref[...], w_ref[...], preferred_element_type=F32,
                precision=lax.Precision.HIGHEST)
    h_ref[...] = h
    p = jnp.dot(h, as_ref[...], preferred_element_type=F32,
                precision=lax.Precision.HIGHEST)
    q = jnp.dot(h, ad_ref[...], preferred_element_type=F32,
                precision=lax.Precision.HIGHEST)
    p_ref[...] = p
    q_ref[...] = q
    pm_ref[...] = jnp.broadcast_to(jnp.max(p, axis=0, keepdims=True), (8, 16))
    qm_ref[...] = jnp.broadcast_to(jnp.max(q, axis=0, keepdims=True), (8, 16))


def _dense_stage(x, W, As, Ad, d_in, d_out):
    return pl.pallas_call(
        _dense_body,
        grid=(GRID,),
        in_specs=[
            pl.BlockSpec((ROWB, d_in), lambda i: (i, 0)),
            pl.BlockSpec((d_in, d_out), lambda i: (0, 0)),
            pl.BlockSpec((d_out, 16), lambda i: (0, 0)),
            pl.BlockSpec((d_out, 16), lambda i: (0, 0)),
        ],
        out_specs=[
            pl.BlockSpec((ROWB, d_out), lambda i: (i, 0)),
            pl.BlockSpec((ROWB, 16), lambda i: (i, 0)),
            pl.BlockSpec((ROWB, 16), lambda i: (i, 0)),
            pl.BlockSpec((8, 16), lambda i: (i, 0)),
            pl.BlockSpec((8, 16), lambda i: (i, 0)),
        ],
        out_shape=[
            jax.ShapeDtypeStruct((N, d_out), F32),
            jax.ShapeDtypeStruct((N, 16), F32),
            jax.ShapeDtypeStruct((N, 16), F32),
            jax.ShapeDtypeStruct((8 * GRID, 16), F32),
            jax.ShapeDtypeStruct((8 * GRID, 16), F32),
        ],
    )(x, W, As, Ad)


def _make_sc_agg(d, wid_cols, BLK):
    NBLK = E // BLK
    BASE_BLKS = NBLK // NW
    EXTRA = NBLK - BASE_BLKS * NW
    nreg = d // L
    mesh = plsc.VectorSubcoreMesh(core_axis_name="c", subcore_axis_name="s")
    cp = pltpu.CompilerParams()
    for fld, val in (("needs_layout_passes", False),
                     ("use_tc_tiling_on_sc", False),
                     ("skip_device_barrier", True)):
        if fld in pltpu.CompilerParams.__dataclass_fields__:
            cp = dataclasses.replace(cp, **{fld: val})

    @functools.partial(
        pl.kernel,
        out_type=jax.ShapeDtypeStruct((NC, N, wid_cols), F32),
        mesh=mesh,
        compiler_params=cp,
        scratch_types=[
            pltpu.VMEM((4, BLK), I32),
            pltpu.VMEM((4, BLK), I32),
            pltpu.VMEM((2, BLK, 16), F32),
            pltpu.VMEM((2, BLK, 16), F32),
            pltpu.VMEM((2, BLK, d), F32),
            pltpu.VMEM((2, BLK, wid_cols), F32),
            pltpu.VMEM((16,), F32),
            pltpu.VMEM_SHARED((N, wid_cols), F32),
            pltpu.SemaphoreType.DMA((4,)),
            pltpu.SemaphoreType.DMA((2,)),
            pltpu.SemaphoreType.DMA((2,)),
        ],
    )
    def agg(h_hbm, p_hbm, q_hbm, src_hbm, dst_hbm, m_hbm, out_hbm,
            sidx, didx, gs, gd, hr, msg, mv, acc,
            isem, gsem, ssem):
        cid = lax.axis_index("c")
        sid = lax.axis_index("s")
        wid = cid * NS + sid

        zero16 = jnp.zeros((16,), F32)

        @pl.loop(0, RCH)
        def _zrow(r):
            for k in range(wid_cols // 16):
                msg[0, r, pl.ds(k * 16, 16)] = zero16

        @pl.loop(0, (NCH + NS - 1) // NS)
        def _zcp(k):
            c = sid + k * NS

            @pl.when(c < NCH)
            def _():
                pltpu.sync_copy(msg.at[0].at[pl.ds(0, RCH)],
                                acc.at[pl.ds(c * RCH, RCH)])

        pltpu.sync_copy(m_hbm, mv)
        plsc.subcore_barrier()

        m = mv[...]
        lane = lax.iota(I32, L)
        div8 = lane // 8

        blk0 = wid * BASE_BLKS + jnp.minimum(wid, EXTRA)

        def valid(j):
            return jnp.logical_or(j < BASE_BLKS,
                                  jnp.logical_and(j < BASE_BLKS + 1,
                                                  wid < EXTRA))

        def start_idx(j, s):
            e0 = (blk0 + j) * BLK
            pltpu.async_copy(src_hbm.at[pl.ds(e0, BLK)], sidx.at[s], isem.at[s])
            pltpu.async_copy(dst_hbm.at[pl.ds(e0, BLK)], didx.at[s], isem.at[s])

        def wait_idx(s):
            pltpu.make_async_copy(src_hbm.at[pl.ds(0, BLK)], sidx.at[s],
                                  isem.at[s]).wait()
            pltpu.make_async_copy(dst_hbm.at[pl.ds(0, BLK)], didx.at[s],
                                  isem.at[s]).wait()

        def start_gathers(b, s):
            pltpu.async_copy(p_hbm.at[sidx.at[s]], gs.at[b], gsem.at[b])
            pltpu.async_copy(q_hbm.at[didx.at[s]], gd.at[b], gsem.at[b])
            pltpu.async_copy(h_hbm.at[sidx.at[s]], hr.at[b], gsem.at[b])

        def wait_gathers(b):
            pltpu.make_async_copy(p_hbm.at[pl.ds(0, BLK)], gs.at[b],
                                  gsem.at[b]).wait()
            pltpu.make_async_copy(q_hbm.at[pl.ds(0, BLK)], gd.at[b],
                                  gsem.at[b]).wait()
            pltpu.make_async_copy(h_hbm.at[pl.ds(0, BLK)], hr.at[b],
                                  gsem.at[b]).wait()

        def start_scatter(b, s):
            pltpu.async_copy(msg.at[b], acc.at[didx.at[s]], ssem.at[b],
                             add=True)

        def wait_scatter(b):
            pltpu.make_async_copy(msg.at[b], acc.at[pl.ds(0, BLK)],
                                  ssem.at[b]).wait()

        def compute(b):
            @plsc.parallel_loop(0, BLK, unroll=4)
            def _edge(r):
                e = gs[b, r, :] + gd[b, r, :]
                e = jnp.where(e >= 0, e, 0.2 * e)
                w = jnp.exp(e - m)
                for k in range(nreg):
                    hk = hr[b, r, pl.ds(k * L, L)]
                    if d == D1:
                        sc = _dyn_gather(w, 2 * k + div8)
                    else:
                        sc = w
                    msg[b, r, pl.ds(k * L, L)] = hk * sc
                if d == D1:
                    wcol = jnp.where(lane < 8, w, 0.0)
                else:
                    wcol = jnp.where(lane == 0, w, 0.0)
                msg[b, r, pl.ds(d, L)] = wcol

        start_idx(0, 0)
        start_idx(1, 1)
        wait_idx(0)
        start_gathers(0, 0)

        @pl.loop(0, (BASE_BLKS + 4) // 4)
        def _grp(g):
            for u in range(4):
                i = g * 4 + u
                b = u % 2
                s = u % 4

                @pl.when(valid(i))
                def _():
                    wait_gathers(b)

                    @pl.when(valid(i + 1))
                    def _():
                        wait_idx((s + 1) % 4)
                        start_gathers(1 - b, (s + 1) % 4)

                    @pl.when(i >= 2)
                    def _():
                        wait_scatter(b)

                    @pl.when(valid(i + 2))
                    def _():
                        start_idx(i + 2, (s + 2) % 4)

                    compute(b)
                    start_scatter(b, s)

        wait_scatter(0)
        wait_scatter(1)
        plsc.subcore_barrier()

        @pl.loop(0, (NCH + NS - 1) // NS)
        def _dump(k):
            c = sid + k * NS

            @pl.when(c < NCH)
            def _():
                pltpu.sync_copy(acc.at[pl.ds(c * RCH, RCH)],
                                out_hbm.at[cid, pl.ds(c * RCH, RCH)])

    return agg


_sc_agg_l1 = _make_sc_agg(D1, D1 + 16, 128)
_sc_agg_l2 = _make_sc_agg(D2, D2 + 16, 64)


def _epi_math(a0, a1, h, p, q, m, b, d, heads, do_elu):
    es = p + q
    es = jnp.where(es >= 0, es, 0.2 * es)
    wself = jnp.exp(es - m)
    li = lax.broadcasted_iota(I32, (16, d), 0)
    ci = lax.broadcasted_iota(I32, (16, d), 1)
    if heads == 8:
        S = (li == ci // 8).astype(F32)
    else:
        S = (li == 0).astype(F32)
    accw = a0[:, d:] + a1[:, d:] + wself
    den = jnp.dot(accw, S, preferred_element_type=F32,
                  precision=lax.Precision.HIGHEST)
    wE = jnp.dot(wself, S, preferred_element_type=F32,
                 precision=lax.Precision.HIGHEST)
    num = a0[:, :d] + a1[:, :d] + wE * h
    o = num / (den + 1e-16) + b
    if do_elu:
        o = jnp.where(o > 0, o, jnp.exp(o) - 1.0)
    return o


def _epi1_dense2_body(a0_ref, a1_ref, h_ref, p_ref, q_ref, m_ref, b_ref,
                      w2_ref, as_ref, ad_ref,
                      h2_ref, p2_ref, q2_ref, pm_ref, qm_ref):
    o = _epi_math(a0_ref[...], a1_ref[...], h_ref[...], p_ref[...],
                  q_ref[...], m_ref[0:1, :], b_ref[0:1, :], D1, 8, True)
    h2 = jnp.dot(o, w2_ref[...], preferred_element_type=F32,
                 precision=lax.Precision.HIGHEST)
    h2_ref[...] = h2
    p2 = jnp.dot(h2, as_ref[...], preferred_element_type=F32,
                 precision=lax.Precision.HIGHEST)
    q2 = jnp.dot(h2, ad_ref[...], preferred_element_type=F32,
                 precision=lax.Precision.HIGHEST)
    p2_ref[...] = p2
    q2_ref[...] = q2
    pm_ref[...] = jnp.broadcast_to(jnp.max(p2, axis=0, keepdims=True), (8, 16))
    qm_ref[...] = jnp.broadcast_to(jnp.max(q2, axis=0, keepdims=True), (8, 16))


def _epi1_dense2(a0, a1, h, p, q, m8, b8, W2, As2, Ad2):
    return pl.pallas_call(
        _epi1_dense2_body,
        grid=(GRID,),
        in_specs=[
            pl.BlockSpec((ROWB, D1 + 16), lambda i: (i, 0)),
            pl.BlockSpec((ROWB, D1 + 16), lambda i: (i, 0)),
            pl.BlockSpec((ROWB, D1), lambda i: (i, 0)),
            pl.BlockSpec((ROWB, 16), lambda i: (i, 0)),
            pl.BlockSpec((ROWB, 16), lambda i: (i, 0)),
            pl.BlockSpec((8, 16), lambda i: (0, 0)),
            pl.BlockSpec((8, D1), lambda i: (0, 0)),
            pl.BlockSpec((D1, D2), lambda i: (0, 0)),
            pl.BlockSpec((D2, 16), lambda i: (0, 0)),
            pl.BlockSpec((D2, 16), lambda i: (0, 0)),
        ],
        out_specs=[
            pl.BlockSpec((ROWB, D2), lambda i: (i, 0)),
            pl.BlockSpec((ROWB, 16), lambda i: (i, 0)),
            pl.BlockSpec((ROWB, 16), lambda i: (i, 0)),
            pl.BlockSpec((8, 16), lambda i: (i, 0)),
            pl.BlockSpec((8, 16), lambda i: (i, 0)),
        ],
        out_shape=[
            jax.ShapeDtypeStruct((N, D2), F32),
            jax.ShapeDtypeStruct((N, 16), F32),
            jax.ShapeDtypeStruct((N, 16), F32),
            jax.ShapeDtypeStruct((8 * GRID, 16), F32),
            jax.ShapeDtypeStruct((8 * GRID, 16), F32),
        ],
    )(a0, a1, h, p, q, m8, b8, W2, As2, Ad2)


def _epi2_pool_body(a0_ref, a1_ref, h_ref, p_ref, q_ref, m_ref, b_ref,
                    bat_ref, o_ref, sums, cnts):
    i = pl.program_id(0)

    @pl.when(i == 0)
    def _():
        sums[...] = jnp.zeros_like(sums)
        cnts[...] = jnp.zeros_like(cnts)

    o = _epi_math(a0_ref[...], a1_ref[...], h_ref[...], p_ref[...],
                  q_ref[...], m_ref[0:1, :], b_ref[0:1, :], D2, 1, False)
    bcol = bat_ref[0]
    for g in range(G):
        sel = (bcol == g)
        msum = jnp.sum(jnp.where(sel, o, 0.0), axis=0, keepdims=True)
        cnt = jnp.sum(jnp.where(sel, 1.0, 0.0))
        sums[pl.ds(g, 1), :] += msum
        cnts[pl.ds(g, 1), :] += jnp.full((1, D2), cnt, F32)

    @pl.when(i == pl.num_programs(0) - 1)
    def _():
        pooled = sums[...] / jnp.maximum(cnts[...], 1.0)
        mx = jnp.max(pooled, axis=1, keepdims=True)
        ex = jnp.exp(pooled - mx)
        lse = jnp.log(jnp.sum(ex, axis=1, keepdims=True))
        o_ref[...] = pooled - mx - lse


def _epi2_pool(a0, a1, h, p, q, m8, b8, batch3):
    return pl.pallas_call(
        _epi2_pool_body,
        grid=(GRID,),
        in_specs=[
            pl.BlockSpec((ROWB, D2 + 16), lambda i: (i, 0)),
            pl.BlockSpec((ROWB, D2 + 16), lambda i: (i, 0)),
            pl.BlockSpec((ROWB, D2), lambda i: (i, 0)),
            pl.BlockSpec((ROWB, 16), lambda i: (i, 0)),
            pl.BlockSpec((ROWB, 16), lambda i: (i, 0)),
            pl.BlockSpec((8, 16), lambda i: (0, 0)),
            pl.BlockSpec((8, D2), lambda i: (0, 0)),
            pl.BlockSpec((1, ROWB, 1), lambda i: (i, 0, 0)),
        ],
        out_specs=pl.BlockSpec((G, D2), lambda i: (0, 0)),
        out_shape=jax.ShapeDtypeStruct((G, D2), F32),
        scratch_shapes=[
            pltpu.VMEM((G, D2), F32),
            pltpu.VMEM((G, D2), F32),
        ],
    )(a0, a1, h, p, q, m8, b8, batch3)


def _expand_attn1(a):
    eye8 = jnp.eye(8, dtype=F32)
    m = (a[:, :, None] * eye8[:, None, :]).reshape(D1, 8)
    return jnp.concatenate([m, m], axis=1)


def _bound16(pm, qm):
    return jnp.maximum(jnp.max(pm, axis=0) + jnp.max(qm, axis=0), 0.0)


def kernel(x, edge_index, batch, W1, a_src1, a_dst1, b1, W2, a_src2,
           a_dst2, b2):
    src = edge_index[0].astype(I32)
    dst = edge_index[1].astype(I32)
    batch3 = batch.astype(I32).reshape(GRID, ROWB, 1)

    As1 = _expand_attn1(a_src1)
    Ad1 = _expand_attn1(a_dst1)
    b1_8 = jnp.broadcast_to(b1[None, :], (8, D1))
    As2 = jnp.tile(a_src2.reshape(D2, 1), (1, 16))
    Ad2 = jnp.tile(a_dst2.reshape(D2, 1), (1, 16))
    b2_8 = jnp.broadcast_to(b2[None, :], (8, D2))

    h1, p1, q1, pm1, qm1 = _dense_stage(x, W1, As1, Ad1, D_IN, D1)
    m1 = _bound16(pm1, qm1)
    acc1 = _sc_agg_l1(h1, p1, q1, src, dst, m1)
    m1_8 = jnp.broadcast_to(m1[None, :], (8, 16))
    h2, p2, q2, pm2, qm2 = _epi1_dense2(acc1[0], acc1[1], h1, p1, q1,
                                        m1_8, b1_8, W2, As2, Ad2)
    m2 = _bound16(pm2, qm2)
    acc2 = _sc_agg_l2(h2, p2, q2, src, dst, m2)
    m2_8 = jnp.broadcast_to(m2[None, :], (8, 16))
    return _epi2_pool(acc2[0], acc2[1], h2, p2, q2, m2_8, b2_8, batch3)


# speedup vs baseline: 1.1622x; 1.1622x over previous
import dataclasses
import functools

import jax
import jax.numpy as jnp
from jax import lax
from jax.experimental import pallas as pl
from jax.experimental.pallas import tpu as pltpu
from jax.experimental.pallas import tpu_sc as plsc

N = 10000
E = 320000
D_IN = 128
H1 = 8
C1 = 8
D1 = H1 * C1
D2 = 128
G = 16

NC, NS, L = 2, 16, 16
NW = NC * NS
ROWB = 1000
GRID = N // ROWB
RCH = 40
NCH = N // RCH

F32 = jnp.float32
I32 = jnp.int32

_GDN = lax.GatherDimensionNumbers(
    offset_dims=(), collapsed_slice_dims=(0,), start_index_map=(0,))


def _dyn_gather(v, idx):
    return lax.gather(v, idx[:, None], _GDN, slice_sizes=(1,),
                      mode=lax.GatherScatterMode.PROMISE_IN_BOUNDS)


def _dense_body(x_ref, w_ref, as_ref, ad_ref, h_ref, p_ref, q_ref,
                pm_ref, qm_ref):
    h = jnp.dot(x_ref[...], w_ref[...], preferred_element_type=F32,
                precision=lax.Precision.DEFAULT)
    h_ref[...] = h
    p = jnp.dot(h, as_ref[...], preferred_element_type=F32,
                precision=lax.Precision.DEFAULT)
    q = jnp.dot(h, ad_ref[...], preferred_element_type=F32,
                precision=lax.Precision.DEFAULT)
    p_ref[...] = p
    q_ref[...] = q
    pm_ref[...] = jnp.broadcast_to(jnp.max(p, axis=0, keepdims=True), (8, 16))
    qm_ref[...] = jnp.broadcast_to(jnp.max(q, axis=0, keepdims=True), (8, 16))


def _dense_stage(x, W, As, Ad, d_in, d_out):
    return pl.pallas_call(
        _dense_body,
        grid=(GRID,),
        in_specs=[
            pl.BlockSpec((ROWB, d_in), lambda i: (i, 0)),
            pl.BlockSpec((d_in, d_out), lambda i: (0, 0)),
            pl.BlockSpec((d_out, 16), lambda i: (0, 0)),
            pl.BlockSpec((d_out, 16), lambda i: (0, 0)),
        ],
        out_specs=[
            pl.BlockSpec((ROWB, d_out), lambda i: (i, 0)),
            pl.BlockSpec((ROWB, 16), lambda i: (i, 0)),
            pl.BlockSpec((ROWB, 16), lambda i: (i, 0)),
            pl.BlockSpec((8, 16), lambda i: (i, 0)),
            pl.BlockSpec((8, 16), lambda i: (i, 0)),
        ],
        out_shape=[
            jax.ShapeDtypeStruct((N, d_out), F32),
            jax.ShapeDtypeStruct((N, 16), F32),
            jax.ShapeDtypeStruct((N, 16), F32),
            jax.ShapeDtypeStruct((8 * GRID, 16), F32),
            jax.ShapeDtypeStruct((8 * GRID, 16), F32),
        ],
    )(x, W, As, Ad)


def _make_sc_agg(d, wid_cols, BLK):
    NBLK = E // BLK
    BASE_BLKS = NBLK // NW
    EXTRA = NBLK - BASE_BLKS * NW
    nreg = d // L
    mesh = plsc.VectorSubcoreMesh(core_axis_name="c", subcore_axis_name="s")
    cp = pltpu.CompilerParams()
    for fld, val in (("needs_layout_passes", False),
                     ("use_tc_tiling_on_sc", False),
                     ("skip_device_barrier", True)):
        if fld in pltpu.CompilerParams.__dataclass_fields__:
            cp = dataclasses.replace(cp, **{fld: val})

    @functools.partial(
        pl.kernel,
        out_type=jax.ShapeDtypeStruct((NC, N, wid_cols), F32),
        mesh=mesh,
        compiler_params=cp,
        scratch_types=[
            pltpu.VMEM((4, BLK), I32),
            pltpu.VMEM((4, BLK), I32),
            pltpu.VMEM((2, BLK, 16), F32),
            pltpu.VMEM((2, BLK, 16), F32),
            pltpu.VMEM((2, BLK, d), F32),
            pltpu.VMEM((2, BLK, wid_cols), F32),
            pltpu.VMEM((16,), F32),
            pltpu.VMEM_SHARED((N, wid_cols), F32),
            pltpu.SemaphoreType.DMA((4,)),
            pltpu.SemaphoreType.DMA((2,)),
            pltpu.SemaphoreType.DMA((2,)),
        ],
    )
    def agg(h_hbm, p_hbm, q_hbm, src_hbm, dst_hbm, m_hbm, out_hbm,
            sidx, didx, gs, gd, hr, msg, mv, acc,
            isem, gsem, ssem):
        cid = lax.axis_index("c")
        sid = lax.axis_index("s")
        wid = cid * NS + sid

        zero16 = jnp.zeros((16,), F32)

        @pl.loop(0, RCH)
        def _zrow(r):
            for k in range(wid_cols // 16):
                msg[0, r, pl.ds(k * 16, 16)] = zero16

        @pl.loop(0, (NCH + NS - 1) // NS)
        def _zcp(k):
            c = sid + k * NS

            @pl.when(c < NCH)
            def _():
                pltpu.sync_copy(msg.at[0].at[pl.ds(0, RCH)],
                                acc.at[pl.ds(c * RCH, RCH)])

        pltpu.sync_copy(m_hbm, mv)
        plsc.subcore_barrier()

        m = mv[...]
        lane = lax.iota(I32, L)
        div8 = lane // 8

        blk0 = wid * BASE_BLKS + jnp.minimum(wid, EXTRA)

        def valid(j):
            return jnp.logical_or(j < BASE_BLKS,
                                  jnp.logical_and(j < BASE_BLKS + 1,
                                                  wid < EXTRA))

        def start_idx(j, s):
            e0 = (blk0 + j) * BLK
            pltpu.async_copy(src_hbm.at[pl.ds(e0, BLK)], sidx.at[s], isem.at[s])
            pltpu.async_copy(dst_hbm.at[pl.ds(e0, BLK)], didx.at[s], isem.at[s])

        def wait_idx(s):
            pltpu.make_async_copy(src_hbm.at[pl.ds(0, BLK)], sidx.at[s],
                                  isem.at[s]).wait()
            pltpu.make_async_copy(dst_hbm.at[pl.ds(0, BLK)], didx.at[s],
                                  isem.at[s]).wait()

        def start_gathers(b, s):
            pltpu.async_copy(p_hbm.at[sidx.at[s]], gs.at[b], gsem.at[b])
            pltpu.async_copy(q_hbm.at[didx.at[s]], gd.at[b], gsem.at[b])
            pltpu.async_copy(h_hbm.at[sidx.at[s]], hr.at[b], gsem.at[b])

        def wait_gathers(b):
            pltpu.make_async_copy(p_hbm.at[pl.ds(0, BLK)], gs.at[b],
                                  gsem.at[b]).wait()
            pltpu.make_async_copy(q_hbm.at[pl.ds(0, BLK)], gd.at[b],
                                  gsem.at[b]).wait()
            pltpu.make_async_copy(h_hbm.at[pl.ds(0, BLK)], hr.at[b],
                                  gsem.at[b]).wait()

        def start_scatter(b, s):
            pltpu.async_copy(msg.at[b], acc.at[didx.at[s]], ssem.at[b],
                             add=True)

        def wait_scatter(b):
            pltpu.make_async_copy(msg.at[b], acc.at[pl.ds(0, BLK)],
                                  ssem.at[b]).wait()

        def compute(b):
            @plsc.parallel_loop(0, BLK, unroll=4)
            def _edge(r):
                e = gs[b, r, :] + gd[b, r, :]
                e = jnp.where(e >= 0, e, 0.2 * e)
                w = jnp.exp(e - m)
                for k in range(nreg):
                    hk = hr[b, r, pl.ds(k * L, L)]
                    if d == D1:
                        sc = _dyn_gather(w, 2 * k + div8)
                    else:
                        sc = w
                    msg[b, r, pl.ds(k * L, L)] = hk * sc
                if d == D1:
                    wcol = jnp.where(lane < 8, w, 0.0)
                else:
                    wcol = jnp.where(lane == 0, w, 0.0)
                msg[b, r, pl.ds(d, L)] = wcol

        start_idx(0, 0)
        start_idx(1, 1)
        wait_idx(0)
        start_gathers(0, 0)

        @pl.loop(0, (BASE_BLKS + 4) // 4)
        def _grp(g):
            for u in range(4):
                i = g * 4 + u
                b = u % 2
                s = u % 4

                @pl.when(valid(i))
                def _():
                    wait_gathers(b)

                    @pl.when(valid(i + 1))
                    def _():
                        wait_idx((s + 1) % 4)
                        start_gathers(1 - b, (s + 1) % 4)

                    @pl.when(i >= 2)
                    def _():
                        wait_scatter(b)

                    @pl.when(valid(i + 2))
                    def _():
                        start_idx(i + 2, (s + 2) % 4)

                    compute(b)
                    start_scatter(b, s)

        wait_scatter(0)
        wait_scatter(1)
        plsc.subcore_barrier()

        @pl.loop(0, (NCH + NS - 1) // NS)
        def _dump(k):
            c = sid + k * NS

            @pl.when(c < NCH)
            def _():
                pltpu.sync_copy(acc.at[pl.ds(c * RCH, RCH)],
                                out_hbm.at[cid, pl.ds(c * RCH, RCH)])

    return agg


_sc_agg_l1 = _make_sc_agg(D1, D1 + 16, 128)
_sc_agg_l2 = _make_sc_agg(D2, D2 + 16, 64)


def _epi_math(a0, a1, h, p, q, m, b, d, heads, do_elu):
    es = p + q
    es = jnp.where(es >= 0, es, 0.2 * es)
    wself = jnp.exp(es - m)
    accw = a0[:, d:] + a1[:, d:] + wself

    def expand(v):
        if heads == 8:
            cols = [jnp.broadcast_to(v[:, hh:hh + 1], (v.shape[0], 8))
                    for hh in range(8)]
            return jnp.concatenate(cols, axis=1)
        return jnp.broadcast_to(v[:, 0:1], (v.shape[0], d))

    den = expand(accw)
    wE = expand(wself)
    num = a0[:, :d] + a1[:, :d] + wE * h
    o = num / (den + 1e-16) + b
    if do_elu:
        o = jnp.where(o > 0, o, jnp.exp(o) - 1.0)
    return o


def _epi1_dense2_body(a_ref, h_ref, p_ref, q_ref, m_ref, b_ref,
                      w2_ref, as_ref, ad_ref,
                      h2_ref, p2_ref, q2_ref, pm_ref, qm_ref):
    o = _epi_math(a_ref[0], a_ref[1], h_ref[...], p_ref[...],
                  q_ref[...], m_ref[0:1, :], b_ref[0:1, :], D1, 8, True)
    h2 = jnp.dot(o, w2_ref[...], preferred_element_type=F32,
                 precision=lax.Precision.DEFAULT)
    h2_ref[...] = h2
    p2 = jnp.dot(h2, as_ref[...], preferred_element_type=F32,
                 precision=lax.Precision.DEFAULT)
    q2 = jnp.dot(h2, ad_ref[...], preferred_element_type=F32,
                 precision=lax.Precision.DEFAULT)
    p2_ref[...] = p2
    q2_ref[...] = q2
    pm_ref[...] = jnp.broadcast_to(jnp.max(p2, axis=0, keepdims=True), (8, 16))
    qm_ref[...] = jnp.broadcast_to(jnp.max(q2, axis=0, keepdims=True), (8, 16))


def _epi1_dense2(a, h, p, q, m8, b8, W2, As2, Ad2):
    return pl.pallas_call(
        _epi1_dense2_body,
        grid=(GRID,),
        in_specs=[
            pl.BlockSpec((NC, ROWB, D1 + 16), lambda i: (0, i, 0)),
            pl.BlockSpec((ROWB, D1), lambda i: (i, 0)),
            pl.BlockSpec((ROWB, 16), lambda i: (i, 0)),
            pl.BlockSpec((ROWB, 16), lambda i: (i, 0)),
            pl.BlockSpec((8, 16), lambda i: (0, 0)),
            pl.BlockSpec((8, D1), lambda i: (0, 0)),
            pl.BlockSpec((D1, D2), lambda i: (0, 0)),
            pl.BlockSpec((D2, 16), lambda i: (0, 0)),
            pl.BlockSpec((D2, 16), lambda i: (0, 0)),
        ],
        out_specs=[
            pl.BlockSpec((ROWB, D2), lambda i: (i, 0)),
            pl.BlockSpec((ROWB, 16), lambda i: (i, 0)),
            pl.BlockSpec((ROWB, 16), lambda i: (i, 0)),
            pl.BlockSpec((8, 16), lambda i: (i, 0)),
            pl.BlockSpec((8, 16), lambda i: (i, 0)),
        ],
        out_shape=[
            jax.ShapeDtypeStruct((N, D2), F32),
            jax.ShapeDtypeStruct((N, 16), F32),
            jax.ShapeDtypeStruct((N, 16), F32),
            jax.ShapeDtypeStruct((8 * GRID, 16), F32),
            jax.ShapeDtypeStruct((8 * GRID, 16), F32),
        ],
    )(a, h, p, q, m8, b8, W2, As2, Ad2)


def _epi2_pool_body(a_ref, h_ref, p_ref, q_ref, m_ref, b_ref,
                    bat_ref, o_ref, sums, cnts):
    i = pl.program_id(0)

    @pl.when(i == 0)
    def _():
        sums[...] = jnp.zeros_like(sums)
        cnts[...] = jnp.zeros_like(cnts)

    o = _epi_math(a_ref[0], a_ref[1], h_ref[...], p_ref[...],
                  q_ref[...], m_ref[0:1, :], b_ref[0:1, :], D2, 1, False)
    bcol = bat_ref[0]
    for g in range(G):
        sel = (bcol == g)
        msum = jnp.sum(jnp.where(sel, o, 0.0), axis=0, keepdims=True)
        cnt = jnp.sum(jnp.where(sel, 1.0, 0.0))
        sums[pl.ds(g, 1), :] += msum
        cnts[pl.ds(g, 1), :] += jnp.full((1, D2), cnt, F32)

    @pl.when(i == pl.num_programs(0) - 1)
    def _():
        pooled = sums[...] / jnp.maximum(cnts[...], 1.0)
        mx = jnp.max(pooled, axis=1, keepdims=True)
        ex = jnp.exp(pooled - mx)
        lse = jnp.log(jnp.sum(ex, axis=1, keepdims=True))
        o_ref[...] = pooled - mx - lse


def _epi2_pool(a, h, p, q, m8, b8, batch3):
    return pl.pallas_call(
        _epi2_pool_body,
        grid=(GRID,),
        in_specs=[
            pl.BlockSpec((NC, ROWB, D2 + 16), lambda i: (0, i, 0)),
            pl.BlockSpec((ROWB, D2), lambda i: (i, 0)),
            pl.BlockSpec((ROWB, 16), lambda i: (i, 0)),
            pl.BlockSpec((ROWB, 16), lambda i: (i, 0)),
            pl.BlockSpec((8, 16), lambda i: (0, 0)),
            pl.BlockSpec((8, D2), lambda i: (0, 0)),
            pl.BlockSpec((1, ROWB, 1), lambda i: (i, 0, 0)),
        ],
        out_specs=pl.BlockSpec((G, D2), lambda i: (0, 0)),
        out_shape=jax.ShapeDtypeStruct((G, D2), F32),
        scratch_shapes=[
            pltpu.VMEM((G, D2), F32),
            pltpu.VMEM((G, D2), F32),
        ],
    )(a, h, p, q, m8, b8, batch3)


def _expand_attn1(a):
    eye8 = jnp.eye(8, dtype=F32)
    m = (a[:, :, None] * eye8[:, None, :]).reshape(D1, 8)
    return jnp.concatenate([m, m], axis=1)


def _bound16(pm, qm):
    return jnp.maximum(jnp.max(pm, axis=0) + jnp.max(qm, axis=0), 0.0)


def kernel(x, edge_index, batch, W1, a_src1, a_dst1, b1, W2, a_src2,
           a_dst2, b2):
    src = edge_index[0].astype(I32)
    dst = edge_index[1].astype(I32)
    batch3 = batch.astype(I32).reshape(GRID, ROWB, 1)

    As1 = _expand_attn1(a_src1)
    Ad1 = _expand_attn1(a_dst1)
    b1_8 = jnp.broadcast_to(b1[None, :], (8, D1))
    As2 = jnp.tile(a_src2.reshape(D2, 1), (1, 16))
    Ad2 = jnp.tile(a_dst2.reshape(D2, 1), (1, 16))
    b2_8 = jnp.broadcast_to(b2[None, :], (8, D2))

    h1, p1, q1, pm1, qm1 = _dense_stage(x, W1, As1, Ad1, D_IN, D1)
    m1 = _bound16(pm1, qm1)
    acc1 = _sc_agg_l1(h1, p1, q1, src, dst, m1)
    m1_8 = jnp.broadcast_to(m1[None, :], (8, 16))
    h2, p2, q2, pm2, qm2 = _epi1_dense2(acc1, h1, p1, q1,
                                        m1_8, b1_8, W2, As2, Ad2)
    m2 = _bound16(pm2, qm2)
    acc2 = _sc_agg_l2(h2, p2, q2, src, dst, m2)
    m2_8 = jnp.broadcast_to(m2[None, :], (8, 16))
    return _epi2_pool(acc2, h2, p2, q2, m2_8, b2_8, batch3)
